# baseline (speedup 1.0000x reference)
import jax
import jax.numpy as jnp
from jax.experimental import pallas as pl

H = 256
MD = 256
EPS = 1e-07
DELTA = 1.1515


def _layer_norm(x, g, b):
    m = x.mean(-1, keepdims=True)
    v = ((x - m) ** 2).mean(-1, keepdims=True)
    return (x - m) / jnp.sqrt(v + 1e-5) * g + b


def _final_add(a_ref, b_ref, o_ref):
    o_ref[...] = a_ref[...] + b_ref[...]


def kernel(x, adj_lists, W_msg, b_msg, ln1_g, ln1_b, Wp, bp, ln2_g, ln2_b,
           Wb1, bb1, Wb2, bb2, alpha_mp, alpha_boom):
    n = x.shape[0]
    ET = adj_lists.shape[0]
    y = _layer_norm(x, ln1_g, ln1_b)
    A = jnp.einsum('nh,thd->tnd', y, W_msg[:, :H, :])
    B = jnp.einsum('nh,thd->tnd', y, W_msg[:, H:, :]) + b_msg[:, None, :]

    sum_acc = jnp.zeros((n, MD), jnp.float32)
    mean_acc = jnp.zeros((n, MD), jnp.float32)
    cnt = jnp.zeros((n,), jnp.float32)
    zmax = jnp.full((n, MD), -jnp.inf, jnp.float32)
    for t in range(ET):
        srcs = adj_lists[t, :, 0]
        tgts = adj_lists[t, :, 1]
        m01 = jax.nn.relu(A[t, srcs, :2 * MD] + B[t, tgts, :2 * MD])
        sum_acc += jax.ops.segment_sum(m01[:, :MD], tgts, num_segments=n)
        mean_acc += jax.ops.segment_sum(m01[:, MD:], tgts, num_segments=n)
        cnt += jax.ops.segment_sum(jnp.ones_like(tgts, jnp.float32), tgts, num_segments=n)
        segmax_a2 = jax.ops.segment_max(A[t, :, 2 * MD:][srcs], tgts, num_segments=n)
        zmax = jnp.maximum(zmax, segmax_a2 + B[t, :, 2 * MD:])
    mean_agg = mean_acc / jnp.maximum(cnt, 1.0)[:, None]
    var_acc = jnp.zeros((n, MD), jnp.float32)
    for t in range(ET):
        srcs = adj_lists[t, :, 0]
        tgts = adj_lists[t, :, 1]
        m1 = jax.nn.relu(A[t, srcs, MD:2 * MD] + B[t, tgts, MD:2 * MD])
        pv = jax.nn.relu(m1 ** 2 - mean_agg[tgts] ** 2) + EPS
        var_acc += jax.ops.segment_sum(pv, tgts, num_segments=n)
    std_agg = jnp.sqrt(var_acc)
    max_agg = jax.nn.relu(zmax)
    agg = jnp.concatenate([sum_acc, mean_agg, std_agg, max_agg], axis=1)
    log_deg = jnp.log(cnt + 1.0)[:, None]
    amp = log_deg / DELTA
    att = DELTA / (log_deg + EPS)
    proj = (agg @ Wp[:4 * MD] + amp * (agg @ Wp[4 * MD:8 * MD])
            + att * (agg @ Wp[8 * MD:]) + bp)
    x1 = x + alpha_mp * proj
    y2 = _layer_norm(x1, ln2_g, ln2_b)
    boom = jax.nn.leaky_relu(y2 @ Wb1 + bb1) @ Wb2 + bb2
    return pl.pallas_call(
        _final_add,
        out_shape=jax.ShapeDtypeStruct((n, H), jnp.float32),
    )(x1, alpha_boom * boom)


# speedup vs baseline: 44.6917x; 44.6917x over previous
import functools

import jax
import jax.numpy as jnp
from jax import lax
from jax.experimental import pallas as pl
from jax.experimental.pallas import tpu as pltpu
from jax.experimental.pallas import tpu_sc as plsc

H = 256
MD = 256
EPS = 1e-07
DELTA = 1.1515
N_BLK = 400

ET = 3
EPT = 53334
NW = 32
C_TILE = 1792
E_PAD = C_TILE * NW
NBH = 10016
NBP = 10048
SENT = 10000


def _wid():
    return lax.axis_index("s") * 2 + lax.axis_index("c")


def _e0():
    return jnp.where(lax.iota(jnp.int32, 16) == 0, 1, 0)


def _cumsum16(v, c32):
    c32[pl.ds(16, 16)] = v
    for st in (1, 2, 4, 8):
        v = v + c32[pl.ds(16 - st, 16)]
        c32[pl.ds(16, 16)] = v
    return v


def _hist_body(tgt_hbm, hist_hbm, tgt_v, hist_v):
    w = _wid()
    for t in range(ET):
        pltpu.sync_copy(tgt_hbm.at[pl.ds(t * E_PAD + w * C_TILE, C_TILE)], tgt_v)

        def zero(i, _):
            hist_v[pl.ds(i * 16, 16)] = jnp.zeros((16,), jnp.int32)
            return 0
        lax.fori_loop(0, NBH // 16, zero, 0)

        e0 = _e0()

        def upd(i, _):
            kv = tgt_v[pl.ds(i * 16, 16)]
            for j in range(16):
                kj = kv[j]
                hist_v[pl.ds(kj, 16)] = hist_v[pl.ds(kj, 16)] + e0
            return 0
        lax.fori_loop(0, C_TILE // 16, upd, 0)
        pltpu.sync_copy(hist_v.at[pl.ds(0, NBH)],
                        hist_hbm.at[pl.ds((t * NW + w) * NBH, NBH)])


def _scat_body(tgt_hbm, src_hbm, hist_hbm, ssrc_hbm, rowptr_hbm,
               tgt_v, src_v, htmp_v, tot_v, off_v, ptr_v, cur_v, sv_v, pos_v,
               c32):
    w = _wid()
    c32[pl.ds(0, 16)] = jnp.zeros((16,), jnp.int32)
    for t in range(ET):
        def zero(i, _):
            tot_v[pl.ds(i * 16, 16)] = jnp.zeros((16,), jnp.int32)
            off_v[pl.ds(i * 16, 16)] = jnp.zeros((16,), jnp.int32)
            return 0
        lax.fori_loop(0, NBH // 16, zero, 0)

        def zero2(i, _):
            ptr_v[pl.ds(i * 16, 16)] = jnp.zeros((16,), jnp.int32)
            return 0
        lax.fori_loop(0, NBP // 16, zero2, 0)

        def merge(wi, _):
            pltpu.sync_copy(hist_hbm.at[pl.ds((t * NW + wi) * NBH, NBH)], htmp_v)
            mine = wi < w

            def acc(i, _):
                h = htmp_v[pl.ds(i * 16, 16)]
                tot_v[pl.ds(i * 16, 16)] += h
                off_v[pl.ds(i * 16, 16)] += jnp.where(mine, h, 0)
                return 0
            return lax.fori_loop(0, NBH // 16, acc, 0)
        lax.fori_loop(0, NW, merge, 0)

        def prefix(i, carry):
            v = tot_v[pl.ds(i * 16, 16)]
            inc = _cumsum16(v, c32)
            ptr_v[pl.ds(i * 16, 16)] = inc - v + carry
            cur_v[pl.ds(i * 16, 16)] = inc - v + carry + off_v[pl.ds(i * 16, 16)]
            return carry + inc[15]
        lax.fori_loop(0, NBH // 16, prefix, jnp.int32(0))

        @pl.when(w == 0)
        def _():
            pltpu.sync_copy(ptr_v, rowptr_hbm.at[pl.ds(t * NBP, NBP)])

        pltpu.sync_copy(tgt_hbm.at[pl.ds(t * E_PAD + w * C_TILE, C_TILE)], tgt_v)
        pltpu.sync_copy(src_hbm.at[pl.ds(t * E_PAD + w * C_TILE, C_TILE)], src_v)

        e0 = _e0()
        iota = lax.iota(jnp.int32, 16)

        def place(r, _):
            for u in range(8):
                i = r * 8 + u
                kv = tgt_v[pl.ds(i * 16, 16)]
                s = src_v[pl.ds(i * 16, 16)]
                pvec = jnp.zeros((16,), jnp.int32)
                for j in range(16):
                    kj = kv[j]
                    win = cur_v[pl.ds(kj, 16)]
                    cur_v[pl.ds(kj, 16)] = win + e0
                    pvec = jnp.where(iota == j, win[0], pvec)
                sv_v[r, pl.ds(u * 16, 16)] = s
                pos_v[r, pl.ds(u * 16, 16)] = pvec + t * E_PAD
            return 0
        lax.fori_loop(0, C_TILE // 128, place, 0)

        for r in range(C_TILE // 128):
            pltpu.sync_copy(sv_v.at[r], ssrc_hbm.at[pos_v.at[r]])


def _sc_sort(tgt_pad, src_pad):
    mesh = plsc.VectorSubcoreMesh(core_axis_name="c", subcore_axis_name="s")
    hist = pl.kernel(
        _hist_body,
        out_type=jax.ShapeDtypeStruct((ET * NW * NBH,), jnp.int32),
        mesh=mesh,
        scratch_types=[pltpu.VMEM((C_TILE,), jnp.int32),
                       pltpu.VMEM((NBH + 16,), jnp.int32)],
        name="sc_hist",
    )(tgt_pad)
    ssrc, rowptr = pl.kernel(
        _scat_body,
        out_type=[jax.ShapeDtypeStruct((ET * E_PAD,), jnp.int32),
                  jax.ShapeDtypeStruct((ET * NBP,), jnp.int32)],
        mesh=mesh,
        scratch_types=[pltpu.VMEM((C_TILE,), jnp.int32),
                       pltpu.VMEM((C_TILE,), jnp.int32),
                       pltpu.VMEM((NBH,), jnp.int32),
                       pltpu.VMEM((NBH,), jnp.int32),
                       pltpu.VMEM((NBH,), jnp.int32),
                       pltpu.VMEM((NBP,), jnp.int32),
                       pltpu.VMEM((NBH + 16,), jnp.int32),
                       pltpu.VMEM((C_TILE // 128, 128), jnp.int32),
                       pltpu.VMEM((C_TILE // 128, 128), jnp.int32),
                       pltpu.VMEM((32,), jnp.int32)],
        name="sc_scatter",
    )(tgt_pad, src_pad, hist)
    return ssrc, rowptr


def _head_body(x_ref, g_ref, b_ref, ws_ref, wt_ref, bc_ref,
               aall_ref, mall_ref, ball_ref):
    x = x_ref[...]
    mu = jnp.mean(x, axis=-1, keepdims=True)
    var = jnp.mean((x - mu) ** 2, axis=-1, keepdims=True)
    y = (x - mu) / jnp.sqrt(var + 1e-5) * g_ref[...] + b_ref[...]
    a = jnp.dot(y, ws_ref[0], preferred_element_type=jnp.float32)
    bm = jnp.dot(y, wt_ref[0], preferred_element_type=jnp.float32) + bc_ref[0]
    aall_ref[...] = a
    mall_ref[...] = a[:, 256:512]
    ball_ref[...] = bm


def _head(x, ln1_g, ln1_b, ws3, wt3, b3):
    n = x.shape[0]
    nb = n // N_BLK
    return pl.pallas_call(
        _head_body,
        grid=(ET * nb,),
        in_specs=[pl.BlockSpec((N_BLK, H), lambda p: (p % nb, 0)),
                  pl.BlockSpec((1, H), lambda p: (0, 0)),
                  pl.BlockSpec((1, H), lambda p: (0, 0)),
                  pl.BlockSpec((1, H, 768), lambda p: (p // nb, 0, 0)),
                  pl.BlockSpec((1, H, 768), lambda p: (p // nb, 0, 0)),
                  pl.BlockSpec((1, 1, 768), lambda p: (p // nb, 0, 0))],
        out_specs=[pl.BlockSpec((N_BLK, 768), lambda p: (p, 0)),
                   pl.BlockSpec((N_BLK, 256), lambda p: (p, 0)),
                   pl.BlockSpec((N_BLK, 768), lambda p: (p, 0))],
        out_shape=[jax.ShapeDtypeStruct((ET * n, 768), jnp.float32),
                   jax.ShapeDtypeStruct((ET * n, 256), jnp.float32),
                   jax.ShapeDtypeStruct((ET * n, 768), jnp.float32)],
    )(x, ln1_g.reshape(1, H), ln1_b.reshape(1, H), ws3, wt3, b3)


NN = 10000
TPB = 313


def _agg_body(rp_hbm, ssrc_hbm, aall_hbm, mall_hbm, bflat_hbm,
              sum_hbm, mean_hbm, var_hbm, zmax_hbm, cnt_hbm,
              rp_v, brow_v, idx_v, rows_v, mrows_v,
              acc_s, acc_m, acc_x, acc_w, zx_v, mu2_v, sem):
    w = _wid()
    lo = w * TPB
    hi = jnp.minimum(lo + TPB, NN)
    loa = (lo >> 3) << 3
    qoff = lo - loa

    def stage_rp(t, _):
        pltpu.sync_copy(rp_hbm.at[pl.ds(pl.multiple_of(t * NBP + loa, 8), 336)],
                        rp_v.at[pl.ds(t * 336, 336)])
        return 0
    lax.fori_loop(0, ET, stage_rp, 0)

    neg = jnp.full((16,), -1e38, jnp.float32)
    zero = jnp.zeros((16,), jnp.float32)

    def node_loop(i, _):
        v = lo + i

        def z16(f, _):
            acc_s[pl.ds(f * 16, 16)] = zero
            acc_m[pl.ds(f * 16, 16)] = zero
            acc_w[pl.ds(f * 16, 16)] = zero
            zx_v[pl.ds(f * 16, 16)] = neg
            return 0
        lax.fori_loop(0, 16, z16, 0)

        def type1(t, cnt):
            win = rp_v[pl.ds(t * 336 + qoff + i, 16)]
            p0 = win[0]
            p1 = win[1]
            pltpu.sync_copy(bflat_hbm.at[pl.ds((t * NN + v) * 768, 768)], brow_v)
            p0a = (p0 >> 3) << 3
            nch = (p1 - p0a + 15) >> 4

            def zx16(f, _):
                acc_x[pl.ds(f * 16, 16)] = neg
                return 0
            lax.fori_loop(0, 16, zx16, 0)

            def chunk(c, _):
                base = p0a + c * 16
                pltpu.sync_copy(
                    ssrc_hbm.at[pl.ds(pl.multiple_of(t * E_PAD + base, 8), 16)],
                    idx_v)
                idx_v[pl.ds(0, 16)] = idx_v[pl.ds(0, 16)] + t * NN
                pltpu.async_copy(aall_hbm.at[idx_v], rows_v, sem).wait()
                s0 = p0 - base
                s1 = p1 - base
                for j in range(16):
                    bad = (s0 > j) | (s1 <= j)

                    @pl.when(bad)
                    def _():
                        def pz(f, _):
                            rows_v[j, pl.ds(f * 16, 16)] = neg
                            return 0
                        lax.fori_loop(0, 48, pz, 0)

                def fs(f, _):
                    cvec = brow_v[pl.ds(f * 16, 16)]
                    acc = acc_s[pl.ds(f * 16, 16)]
                    for j in range(16):
                        acc = acc + jnp.maximum(
                            rows_v[j, pl.ds(f * 16, 16)] + cvec, 0.0)
                    acc_s[pl.ds(f * 16, 16)] = acc
                    return 0
                lax.fori_loop(0, 16, fs, 0)

                def fm(f, _):
                    cvec = brow_v[pl.ds(256 + f * 16, 16)]
                    acc = acc_m[pl.ds(f * 16, 16)]
                    for j in range(16):
                        acc = acc + jnp.maximum(
                            rows_v[j, pl.ds(256 + f * 16, 16)] + cvec, 0.0)
                    acc_m[pl.ds(f * 16, 16)] = acc
                    return 0
                lax.fori_loop(0, 16, fm, 0)

                def fx(f, _):
                    acc = acc_x[pl.ds(f * 16, 16)]
                    for j in range(16):
                        acc = jnp.maximum(acc, rows_v[j, pl.ds(512 + f * 16, 16)])
                    acc_x[pl.ds(f * 16, 16)] = acc
                    return 0
                lax.fori_loop(0, 16, fx, 0)
                return 0
            lax.fori_loop(0, nch, chunk, 0)

            def zfold(f, _):
                zx_v[pl.ds(f * 16, 16)] = jnp.maximum(
                    zx_v[pl.ds(f * 16, 16)],
                    acc_x[pl.ds(f * 16, 16)] + brow_v[pl.ds(512 + f * 16, 16)])
                return 0
            lax.fori_loop(0, 16, zfold, 0)
            return cnt + (p1 - p0)
        cnt_i = lax.fori_loop(0, ET, type1, jnp.int32(0))
        cntf = cnt_i.astype(jnp.float32)

        def fmu(f, _):
            s = acc_m[pl.ds(f * 16, 16)]
            mu = s / jnp.maximum(cntf, 1.0)
            acc_m[pl.ds(f * 16, 16)] = mu
            mu2_v[pl.ds(f * 16, 16)] = mu * mu
            return 0
        lax.fori_loop(0, 16, fmu, 0)

        def type2(t, _):
            win = rp_v[pl.ds(t * 336 + qoff + i, 16)]
            p0 = win[0]
            p1 = win[1]
            pltpu.sync_copy(bflat_hbm.at[pl.ds((t * NN + v) * 768 + 256, 256)],
                            brow_v.at[pl.ds(0, 256)])
            p0a = (p0 >> 3) << 3
            nch = (p1 - p0a + 15) >> 4

            def chunk2(c, _):
                base = p0a + c * 16
                pltpu.sync_copy(
                    ssrc_hbm.at[pl.ds(pl.multiple_of(t * E_PAD + base, 8), 16)],
                    idx_v)
                idx_v[pl.ds(0, 16)] = idx_v[pl.ds(0, 16)] + t * NN
                pltpu.async_copy(mall_hbm.at[idx_v], mrows_v, sem).wait()
                s0 = p0 - base
                s1 = p1 - base
                for j in range(16):
                    bad = (s0 > j) | (s1 <= j)

                    @pl.when(bad)
                    def _():
                        def pz(f, _):
                            mrows_v[j, pl.ds(f * 16, 16)] = neg
                            return 0
                        lax.fori_loop(0, 16, pz, 0)

                def fv(f, _):
                    cvec = brow_v[pl.ds(f * 16, 16)]
                    mq = mu2_v[pl.ds(f * 16, 16)]
                    acc = acc_w[pl.ds(f * 16, 16)]
                    for j in range(16):
                        m1 = jnp.maximum(mrows_v[j, pl.ds(f * 16, 16)] + cvec, 0.0)
                        acc = acc + jnp.maximum(m1 * m1 - mq, 0.0)
                    acc_w[pl.ds(f * 16, 16)] = acc
                    return 0
                lax.fori_loop(0, 16, fv, 0)
                return 0
            lax.fori_loop(0, nch, chunk2, 0)
            return 0
        lax.fori_loop(0, ET, type2, 0)

        def fvar(f, _):
            acc_w[pl.ds(f * 16, 16)] = acc_w[pl.ds(f * 16, 16)] + cntf * EPS
            return 0
        lax.fori_loop(0, 16, fvar, 0)
        mu2_v[pl.ds(0, 16)] = jnp.zeros((16,), jnp.float32) + cntf

        pltpu.sync_copy(acc_s, sum_hbm.at[pl.ds(v * 256, 256)])
        pltpu.sync_copy(acc_m, mean_hbm.at[pl.ds(v * 256, 256)])
        pltpu.sync_copy(acc_w, var_hbm.at[pl.ds(v * 256, 256)])
        pltpu.sync_copy(zx_v, zmax_hbm.at[pl.ds(v * 256, 256)])
        pltpu.sync_copy(mu2_v.at[pl.ds(0, 16)], cnt_hbm.at[pl.ds(v * 16, 16)])
        return 0
    lax.fori_loop(0, hi - lo, node_loop, 0)


def _agg(rowptr, ssrc, aall, mall, bflat):
    mesh = plsc.VectorSubcoreMesh(core_axis_name="c", subcore_axis_name="s")
    return pl.kernel(
        _agg_body,
        out_type=[jax.ShapeDtypeStruct((NN * 256,), jnp.float32),
                  jax.ShapeDtypeStruct((NN * 256,), jnp.float32),
                  jax.ShapeDtypeStruct((NN * 256,), jnp.float32),
                  jax.ShapeDtypeStruct((NN * 256,), jnp.float32),
                  jax.ShapeDtypeStruct((NN * 16,), jnp.float32)],
        mesh=mesh,
        scratch_types=[pltpu.VMEM((ET * 336,), jnp.int32),
                       pltpu.VMEM((768,), jnp.float32),
                       pltpu.VMEM((16,), jnp.int32),
                       pltpu.VMEM((16, 768), jnp.float32),
                       pltpu.VMEM((16, 256), jnp.float32),
                       pltpu.VMEM((256,), jnp.float32),
                       pltpu.VMEM((256,), jnp.float32),
                       pltpu.VMEM((256,), jnp.float32),
                       pltpu.VMEM((256,), jnp.float32),
                       pltpu.VMEM((256,), jnp.float32),
                       pltpu.VMEM((256,), jnp.float32),
                       pltpu.SemaphoreType.DMA],
        name="sc_agg",
    )(rowptr, ssrc, aall, mall, bflat)


def _tail_body(x_ref, sum_ref, mean_ref, var_ref, zmax_ref, cnt_ref,
               wcat_ref, bp_ref, g2_ref, b2_ref, wb1_ref, bb1_ref,
               wb2_ref, bb2_ref, o_ref):
    x = x_ref[...]
    cnt = cnt_ref[...]
    std = jnp.sqrt(var_ref[...])
    mx = jnp.maximum(zmax_ref[...], 0.0)
    agg = jnp.concatenate([sum_ref[...], mean_ref[...], std, mx], axis=1)
    p = jnp.dot(agg, wcat_ref[...], preferred_element_type=jnp.float32)
    log_deg = jnp.log(cnt + 1.0)
    amp = log_deg / DELTA
    att = DELTA / (log_deg + EPS)
    x1 = x + p[:, :256] + amp * p[:, 256:512] + att * p[:, 512:] + bp_ref[...]
    mu = jnp.mean(x1, axis=-1, keepdims=True)
    v = jnp.mean((x1 - mu) ** 2, axis=-1, keepdims=True)
    y2 = (x1 - mu) / jnp.sqrt(v + 1e-5) * g2_ref[...] + b2_ref[...]
    h = jnp.dot(y2, wb1_ref[...], preferred_element_type=jnp.float32) + bb1_ref[...]
    h = jnp.where(h > 0, h, 0.01 * h)
    boom = jnp.dot(h, wb2_ref[...], preferred_element_type=jnp.float32) + bb2_ref[...]
    o_ref[...] = x1 + boom


def _tail(x, sum_acc, mean_agg, var_acc, zmax, cnt,
          wcat, bp_s, ln2_g, ln2_b, wb1, bb1, wb2s, bb2s):
    n = x.shape[0]
    grid = n // N_BLK
    blk = lambda w: pl.BlockSpec((N_BLK, w), lambda i: (i, 0))
    full = lambda s: pl.BlockSpec(s, lambda i: tuple(0 for _ in s))
    return pl.pallas_call(
        _tail_body,
        grid=(grid,),
        in_specs=[blk(H), blk(MD), blk(MD), blk(MD), blk(MD), blk(1),
                  full((1024, 768)), full((1, H)), full((1, H)), full((1, H)),
                  full((H, 1024)), full((1, 1024)), full((1024, H)), full((1, H))],
        out_specs=blk(H),
        out_shape=jax.ShapeDtypeStruct((n, H), jnp.float32),
    )(x, sum_acc, mean_agg, var_acc, zmax, cnt, wcat, bp_s.reshape(1, H),
      ln2_g.reshape(1, H), ln2_b.reshape(1, H), wb1, bb1.reshape(1, 1024),
      wb2s, bb2s.reshape(1, H))


def kernel(x, adj_lists, W_msg, b_msg, ln1_g, ln1_b, Wp, bp, ln2_g, ln2_b,
           Wb1, bb1, Wb2, bb2, alpha_mp, alpha_boom):
    n = x.shape[0]
    ws3 = W_msg[:, :H, :]
    wt3 = W_msg[:, H:, :]
    b3 = b_msg.reshape(ET, 1, 768)
    wcat = alpha_mp * jnp.concatenate(
        [Wp[:4 * MD], Wp[4 * MD:8 * MD], Wp[8 * MD:]], axis=1)
    bp_s = alpha_mp * bp
    wb2s = alpha_boom * Wb2
    bb2s = alpha_boom * bb2

    src_pad = jnp.concatenate(
        [adj_lists[:, :, 0], jnp.zeros((ET, E_PAD - EPT), jnp.int32)], axis=1)
    tgt_pad = jnp.concatenate(
        [adj_lists[:, :, 1], jnp.full((ET, E_PAD - EPT), SENT, jnp.int32)], axis=1)
    ssrc, rowptr = _sc_sort(tgt_pad.reshape(-1), src_pad.reshape(-1))

    aall, mall, ball = _head(x, ln1_g, ln1_b, ws3, wt3, b3)
    sumf, meanf, varf, zmaxf, cntf = _agg(rowptr, ssrc, aall, mall,
                                          ball.reshape(-1))
    return _tail(x, sumf.reshape(n, 256), meanf.reshape(n, 256),
                 varf.reshape(n, 256), zmaxf.reshape(n, 256),
                 cntf.reshape(n, 16)[:, :1],
                 wcat, bp_s, ln2_g, ln2_b, Wb1, bb1, wb2s, bb2s)
